# baseline (speedup 1.0000x reference)
import functools

import jax
import jax.numpy as jnp
from jax import lax
from jax.experimental import pallas as pl
from jax.experimental.pallas import tpu as pltpu
from jax.experimental.pallas import tpu_sc as plsc

N = 10000
E = 320000
D = 128
NC = 2
NS = 16
NW = NC * NS
EW = E // NW
C = 80
NCH = EW // C
NPAD = 10240
DEG_SUB = NPAD // NS
NROW = 10240
ROW_SUB = NROW // NS
ZROWS = 128

_MESH = dict(core_axis_name="c", subcore_axis_name="s")


def _sc_degree(dst3):
    @functools.partial(
        pl.kernel,
        mesh=plsc.VectorSubcoreMesh(**_MESH),
        out_type=jax.ShapeDtypeStruct((NC, NPAD), jnp.float32),
        scratch_types=[
            pltpu.VMEM((NCH, C), jnp.int32),
            pltpu.VMEM((C,), jnp.float32),
            pltpu.VMEM((DEG_SUB,), jnp.float32),
            pltpu.VMEM_SHARED((NPAD,), jnp.float32),
            pltpu.SemaphoreType.DMA,
        ],
    )
    def deg_kernel(dst_hbm, out_hbm, dstv, onesv, zerov, dega, sem):
        cid = lax.axis_index("c")
        sid = lax.axis_index("s")
        wid = cid * NS + sid
        ones16 = jnp.ones((16,), jnp.float32)
        zeros16 = jnp.zeros((16,), jnp.float32)
        for i in range(C // 16):
            onesv[pl.ds(i * 16, 16)] = ones16
        for i in range(DEG_SUB // 16):
            zerov[pl.ds(i * 16, 16)] = zeros16
        pltpu.sync_copy(zerov, dega.at[pl.ds(sid * DEG_SUB, DEG_SUB)])
        pltpu.sync_copy(dst_hbm.at[wid], dstv)
        plsc.subcore_barrier()

        def fire(j, carry):
            pltpu.async_copy(onesv, dega.at[dstv.at[j]], sem, add=True)
            return carry

        def drain(j, carry):
            pltpu.make_async_copy(onesv, dega.at[dstv.at[j]], sem).wait()
            return carry

        lax.fori_loop(0, NCH, fire, 0)
        lax.fori_loop(0, NCH, drain, 0)
        plsc.subcore_barrier()
        sl = pl.ds(sid * DEG_SUB, DEG_SUB)
        pltpu.sync_copy(dega.at[sl], out_hbm.at[cid, sl])

    return deg_kernel(dst3)


def _sc_scatter(y, srcflat, dst3, zrows):
    @functools.partial(
        pl.kernel,
        mesh=plsc.VectorSubcoreMesh(**_MESH),
        out_type=jax.ShapeDtypeStruct((NC, NROW, D), jnp.float32),
        scratch_types=[
            pltpu.VMEM((EW,), jnp.int32),
            pltpu.VMEM((NCH, C), jnp.int32),
            pltpu.VMEM((C, D), jnp.float32),
            pltpu.VMEM((C, D), jnp.float32),
            pltpu.VMEM_SHARED((NROW, D), jnp.float32),
            pltpu.SemaphoreType.DMA,
            pltpu.SemaphoreType.DMA,
            pltpu.SemaphoreType.DMA,
            pltpu.SemaphoreType.DMA,
        ],
    )
    def scat_kernel(y_hbm, src_hbm, dst_hbm, z_hbm, out_hbm,
                    srcv, dstv, rows0, rows1, acc, sem0, sem1, ssem0, ssem1):
        cid = lax.axis_index("c")
        sid = lax.axis_index("s")
        wid = cid * NS + sid
        base = sid * ROW_SUB
        for k in range(ROW_SUB // ZROWS):
            sl = pl.ds(base + k * ZROWS, ZROWS)

            @pl.when(cid == 0)
            def _():
                pltpu.sync_copy(y_hbm.at[sl], acc.at[sl])

            @pl.when(cid != 0)
            def _():
                pltpu.sync_copy(z_hbm, acc.at[sl])

        pltpu.sync_copy(src_hbm.at[pl.ds(wid * EW, EW)], srcv)
        pltpu.sync_copy(dst_hbm.at[wid], dstv)
        plsc.subcore_barrier()

        def g_start(j, buf, sem):
            pltpu.async_copy(y_hbm.at[srcv.at[pl.ds(j * C, C)]], buf, sem)

        def g_wait(j, buf, sem):
            pltpu.make_async_copy(y_hbm.at[srcv.at[pl.ds(j * C, C)]], buf,
                                  sem).wait()

        def s_start(j, buf, sem):
            pltpu.async_copy(buf, acc.at[dstv.at[j]], sem, add=True)

        def s_wait(j, buf, sem):
            pltpu.make_async_copy(buf, acc.at[dstv.at[j]], sem).wait()

        g_start(0, rows0, sem0)
        g_wait(0, rows0, sem0)
        s_start(0, rows0, ssem0)
        g_start(1, rows1, sem1)

        def body(i, carry):
            j = 2 * i
            g_wait(j + 1, rows1, sem1)
            s_start(j + 1, rows1, ssem1)
            s_wait(j, rows0, ssem0)
            g_start(j + 2, rows0, sem0)
            g_wait(j + 2, rows0, sem0)
            s_start(j + 2, rows0, ssem0)
            s_wait(j + 1, rows1, ssem1)
            g_start(j + 3, rows1, sem1)
            return carry

        lax.fori_loop(0, (NCH - 3) // 2, body, 0)
        g_wait(NCH - 2, rows1, sem1)
        s_start(NCH - 2, rows1, ssem1)
        s_wait(NCH - 3, rows0, ssem0)
        g_start(NCH - 1, rows0, sem0)
        g_wait(NCH - 1, rows0, sem0)
        s_start(NCH - 1, rows0, ssem0)
        s_wait(NCH - 2, rows1, ssem1)
        s_wait(NCH - 1, rows0, ssem0)
        plsc.subcore_barrier()
        for k in range(ROW_SUB // ZROWS):
            sl = pl.ds(base + k * ZROWS, ZROWS)
            pltpu.sync_copy(acc.at[sl], out_hbm.at[cid, sl])

    return scat_kernel(y, srcflat, dst3, zrows)


def _tc_matmul(x, W1):
    BLK = 1000

    def body(x_ref, w_ref, o_ref):
        o_ref[...] = jnp.dot(x_ref[...], w_ref[...],
                             preferred_element_type=jnp.float32)

    return pl.pallas_call(
        body,
        grid=(N // BLK,),
        in_specs=[
            pl.BlockSpec((BLK, D), lambda i: (i, 0)),
            pl.BlockSpec((D, D), lambda i: (0, 0)),
        ],
        out_specs=pl.BlockSpec((BLK, D), lambda i: (i, 0)),
        out_shape=jax.ShapeDtypeStruct((NROW, D), jnp.float32),
    )(x, W1)


def _tc_stage1(xw, d0, d1):
    BLK = 1000

    def body(xw_ref, d0_ref, d1_ref, y_ref):
        dis = lax.rsqrt(d0_ref[...] + d1_ref[...] + 1.0)
        y_ref[...] = xw_ref[...] * dis

    return pl.pallas_call(
        body,
        grid=(N // BLK,),
        in_specs=[
            pl.BlockSpec((BLK, D), lambda i: (i, 0)),
            pl.BlockSpec((BLK, 1), lambda i: (i, 0)),
            pl.BlockSpec((BLK, 1), lambda i: (i, 0)),
        ],
        out_specs=pl.BlockSpec((BLK, D), lambda i: (i, 0)),
        out_shape=jax.ShapeDtypeStruct((NROW, D), jnp.float32),
    )(xw, d0, d1)


def _tc_stage2(agg, d0, d1, b1, W2):
    BLK = 1000

    def body(p0_ref, p1_ref, d0_ref, d1_ref, b1_ref, w_ref, y2_ref):
        dis = lax.rsqrt(d0_ref[...] + d1_ref[...] + 1.0)
        h = dis * (p0_ref[0] + p1_ref[0]) + b1_ref[...]
        h = jnp.maximum(h, 0.0)
        y2_ref[...] = jnp.dot(h, w_ref[...], preferred_element_type=jnp.float32) * dis

    return pl.pallas_call(
        body,
        grid=(N // BLK,),
        in_specs=[
            pl.BlockSpec((1, BLK, D), lambda i: (0, i, 0)),
            pl.BlockSpec((1, BLK, D), lambda i: (1, i, 0)),
            pl.BlockSpec((BLK, 1), lambda i: (i, 0)),
            pl.BlockSpec((BLK, 1), lambda i: (i, 0)),
            pl.BlockSpec((1, D), lambda i: (0, 0)),
            pl.BlockSpec((D, D), lambda i: (0, 0)),
        ],
        out_specs=pl.BlockSpec((BLK, D), lambda i: (i, 0)),
        out_shape=jax.ShapeDtypeStruct((NROW, D), jnp.float32),
    )(agg, agg, d0, d1, b1, W2)


def _tc_stage3(agg, d0, d1, b2):
    BLK = 1000

    def body(q0_ref, q1_ref, d0_ref, d1_ref, b2_ref, o_ref):
        dis = lax.rsqrt(d0_ref[...] + d1_ref[...] + 1.0)
        o_ref[...] = dis * (q0_ref[0] + q1_ref[0]) + b2_ref[...]

    return pl.pallas_call(
        body,
        grid=(N // BLK,),
        in_specs=[
            pl.BlockSpec((1, BLK, D), lambda i: (0, i, 0)),
            pl.BlockSpec((1, BLK, D), lambda i: (1, i, 0)),
            pl.BlockSpec((BLK, 1), lambda i: (i, 0)),
            pl.BlockSpec((BLK, 1), lambda i: (i, 0)),
            pl.BlockSpec((1, D), lambda i: (0, 0)),
        ],
        out_specs=pl.BlockSpec((BLK, D), lambda i: (i, 0)),
        out_shape=jax.ShapeDtypeStruct((N, D), jnp.float32),
    )(agg, agg, d0, d1, b2)


def kernel(x, edge_index, W1, b1, W2, b2):
    srcflat = edge_index[0].astype(jnp.int32).reshape(-1)
    dst3 = edge_index[1].astype(jnp.int32).reshape(NW, NCH, C)
    zrows = jnp.zeros((ZROWS, D), jnp.float32)

    xw1 = _tc_matmul(x, W1)
    degp = _sc_degree(dst3)
    d0 = degp[0, :N].reshape(N, 1)
    d1 = degp[1, :N].reshape(N, 1)
    y1 = _tc_stage1(xw1, d0, d1)
    agg1 = _sc_scatter(y1, srcflat, dst3, zrows)
    y2 = _tc_stage2(agg1, d0, d1, b1.reshape(1, D), W2)
    agg2 = _sc_scatter(y2, srcflat, dst3, zrows)
    return _tc_stage3(agg2, d0, d1, b2.reshape(1, D))


# speedup vs baseline: 1.0047x; 1.0047x over previous
import functools

import jax
import jax.numpy as jnp
from jax import lax
from jax.experimental import pallas as pl
from jax.experimental.pallas import tpu as pltpu
from jax.experimental.pallas import tpu_sc as plsc

N = 10000
E = 320000
D = 128
NC = 2
NS = 16
NW = NC * NS
EW = E // NW
C = 80
NCH = EW // C
NPAD = 10240
DEG_SUB = NPAD // NS
NROW = 10240
ROW_SUB = NROW // NS
ZROWS = 128

_MESH = dict(core_axis_name="c", subcore_axis_name="s")


def _sc_degree(dst3):
    @functools.partial(
        pl.kernel,
        mesh=plsc.VectorSubcoreMesh(**_MESH),
        out_type=jax.ShapeDtypeStruct((NC, NPAD), jnp.float32),
        scratch_types=[
            pltpu.VMEM((NCH, C), jnp.int32),
            pltpu.VMEM((C,), jnp.float32),
            pltpu.VMEM((DEG_SUB,), jnp.float32),
            pltpu.VMEM_SHARED((NPAD,), jnp.float32),
            pltpu.SemaphoreType.DMA,
        ],
    )
    def deg_kernel(dst_hbm, out_hbm, dstv, onesv, zerov, dega, sem):
        cid = lax.axis_index("c")
        sid = lax.axis_index("s")
        wid = cid * NS + sid
        ones16 = jnp.ones((16,), jnp.float32)
        zeros16 = jnp.zeros((16,), jnp.float32)
        for i in range(C // 16):
            onesv[pl.ds(i * 16, 16)] = ones16
        for i in range(DEG_SUB // 16):
            zerov[pl.ds(i * 16, 16)] = zeros16
        pltpu.sync_copy(zerov, dega.at[pl.ds(sid * DEG_SUB, DEG_SUB)])
        pltpu.sync_copy(dst_hbm.at[wid], dstv)
        plsc.subcore_barrier()

        def fire(j, carry):
            pltpu.async_copy(onesv, dega.at[dstv.at[j]], sem, add=True)
            return carry

        def drain(j, carry):
            pltpu.make_async_copy(onesv, dega.at[dstv.at[j]], sem).wait()
            return carry

        lax.fori_loop(0, NCH, fire, 0)
        lax.fori_loop(0, NCH, drain, 0)
        plsc.subcore_barrier()
        sl = pl.ds(sid * DEG_SUB, DEG_SUB)
        pltpu.sync_copy(dega.at[sl], out_hbm.at[cid, sl])

    return deg_kernel(dst3)


def _sc_scatter(y, srcflat, dst3, zrows):
    @functools.partial(
        pl.kernel,
        mesh=plsc.VectorSubcoreMesh(**_MESH),
        out_type=jax.ShapeDtypeStruct((NC, NROW, D), jnp.float32),
        scratch_types=[
            pltpu.VMEM((EW,), jnp.int32),
            pltpu.VMEM((NCH, C), jnp.int32),
            pltpu.VMEM((C, D), jnp.float32),
            pltpu.VMEM((C, D), jnp.float32),
            pltpu.VMEM_SHARED((NROW, D), jnp.float32),
            pltpu.SemaphoreType.DMA,
            pltpu.SemaphoreType.DMA,
            pltpu.SemaphoreType.DMA,
            pltpu.SemaphoreType.DMA,
        ],
    )
    def scat_kernel(y_hbm, src_hbm, dst_hbm, z_hbm, out_hbm,
                    srcv, dstv, rows0, rows1, acc, sem0, sem1, ssem0, ssem1):
        cid = lax.axis_index("c")
        sid = lax.axis_index("s")
        wid = cid * NS + sid
        base = sid * ROW_SUB
        for k in range(ROW_SUB // ZROWS):
            sl = pl.ds(base + k * ZROWS, ZROWS)

            @pl.when(cid == 0)
            def _():
                pltpu.sync_copy(y_hbm.at[sl], acc.at[sl])

            @pl.when(cid != 0)
            def _():
                pltpu.sync_copy(z_hbm, acc.at[sl])

        pltpu.sync_copy(src_hbm.at[pl.ds(wid * EW, EW)], srcv)
        pltpu.sync_copy(dst_hbm.at[wid], dstv)
        plsc.subcore_barrier()

        def g_start(j, buf, sem):
            pltpu.async_copy(y_hbm.at[srcv.at[pl.ds(j * C, C)]], buf, sem)

        def g_wait(j, buf, sem):
            pltpu.make_async_copy(y_hbm.at[srcv.at[pl.ds(j * C, C)]], buf,
                                  sem).wait()

        def s_start(j, buf, sem):
            pltpu.async_copy(buf, acc.at[dstv.at[j]], sem, add=True)

        def s_wait(j, buf, sem):
            pltpu.make_async_copy(buf, acc.at[dstv.at[j]], sem).wait()

        g_start(0, rows0, sem0)
        g_wait(0, rows0, sem0)
        s_start(0, rows0, ssem0)
        g_start(1, rows1, sem1)

        def body(i, carry):
            j = 2 * i
            g_wait(j + 1, rows1, sem1)
            s_start(j + 1, rows1, ssem1)
            s_wait(j, rows0, ssem0)
            g_start(j + 2, rows0, sem0)
            g_wait(j + 2, rows0, sem0)
            s_start(j + 2, rows0, ssem0)
            s_wait(j + 1, rows1, ssem1)
            g_start(j + 3, rows1, sem1)
            return carry

        lax.fori_loop(0, (NCH - 3) // 2, body, 0)
        g_wait(NCH - 2, rows1, sem1)
        s_start(NCH - 2, rows1, ssem1)
        s_wait(NCH - 3, rows0, ssem0)
        g_start(NCH - 1, rows0, sem0)
        g_wait(NCH - 1, rows0, sem0)
        s_start(NCH - 1, rows0, ssem0)
        s_wait(NCH - 2, rows1, ssem1)
        s_wait(NCH - 1, rows0, ssem0)
        plsc.subcore_barrier()
        for k in range(ROW_SUB // ZROWS):
            sl = pl.ds(base + k * ZROWS, ZROWS)
            pltpu.sync_copy(acc.at[sl], out_hbm.at[cid, sl])

    return scat_kernel(y, srcflat, dst3, zrows)


def _tc_stage1(x, W1, d0, d1):
    BLK = 1000

    def body(x_ref, w_ref, d0_ref, d1_ref, y_ref):
        dis = lax.rsqrt(d0_ref[...] + d1_ref[...] + 1.0)
        xw = jnp.dot(x_ref[...], w_ref[...], preferred_element_type=jnp.float32)
        y_ref[...] = xw * dis

    return pl.pallas_call(
        body,
        grid=(N // BLK,),
        in_specs=[
            pl.BlockSpec((BLK, D), lambda i: (i, 0)),
            pl.BlockSpec((D, D), lambda i: (0, 0)),
            pl.BlockSpec((BLK, 1), lambda i: (i, 0)),
            pl.BlockSpec((BLK, 1), lambda i: (i, 0)),
        ],
        out_specs=pl.BlockSpec((BLK, D), lambda i: (i, 0)),
        out_shape=jax.ShapeDtypeStruct((NROW, D), jnp.float32),
    )(x, W1, d0, d1)


def _tc_stage2(agg, d0, d1, b1, W2):
    BLK = 1000

    def body(p0_ref, p1_ref, d0_ref, d1_ref, b1_ref, w_ref, y2_ref):
        dis = lax.rsqrt(d0_ref[...] + d1_ref[...] + 1.0)
        h = dis * (p0_ref[0] + p1_ref[0]) + b1_ref[...]
        h = jnp.maximum(h, 0.0)
        y2_ref[...] = jnp.dot(h, w_ref[...], preferred_element_type=jnp.float32) * dis

    return pl.pallas_call(
        body,
        grid=(N // BLK,),
        in_specs=[
            pl.BlockSpec((1, BLK, D), lambda i: (0, i, 0)),
            pl.BlockSpec((1, BLK, D), lambda i: (1, i, 0)),
            pl.BlockSpec((BLK, 1), lambda i: (i, 0)),
            pl.BlockSpec((BLK, 1), lambda i: (i, 0)),
            pl.BlockSpec((1, D), lambda i: (0, 0)),
            pl.BlockSpec((D, D), lambda i: (0, 0)),
        ],
        out_specs=pl.BlockSpec((BLK, D), lambda i: (i, 0)),
        out_shape=jax.ShapeDtypeStruct((NROW, D), jnp.float32),
    )(agg, agg, d0, d1, b1, W2)


def _tc_stage3(agg, d0, d1, b2):
    BLK = 1000

    def body(q0_ref, q1_ref, d0_ref, d1_ref, b2_ref, o_ref):
        dis = lax.rsqrt(d0_ref[...] + d1_ref[...] + 1.0)
        o_ref[...] = dis * (q0_ref[0] + q1_ref[0]) + b2_ref[...]

    return pl.pallas_call(
        body,
        grid=(N // BLK,),
        in_specs=[
            pl.BlockSpec((1, BLK, D), lambda i: (0, i, 0)),
            pl.BlockSpec((1, BLK, D), lambda i: (1, i, 0)),
            pl.BlockSpec((BLK, 1), lambda i: (i, 0)),
            pl.BlockSpec((BLK, 1), lambda i: (i, 0)),
            pl.BlockSpec((1, D), lambda i: (0, 0)),
        ],
        out_specs=pl.BlockSpec((BLK, D), lambda i: (i, 0)),
        out_shape=jax.ShapeDtypeStruct((N, D), jnp.float32),
    )(agg, agg, d0, d1, b2)


def kernel(x, edge_index, W1, b1, W2, b2):
    srcflat = edge_index[0].astype(jnp.int32).reshape(-1)
    dst3 = edge_index[1].astype(jnp.int32).reshape(NW, NCH, C)
    zrows = jnp.zeros((ZROWS, D), jnp.float32)

    degp = _sc_degree(dst3)
    d0 = degp[0, :N].reshape(N, 1)
    d1 = degp[1, :N].reshape(N, 1)
    y1 = _tc_stage1(x, W1, d0, d1)
    agg1 = _sc_scatter(y1, srcflat, dst3, zrows)
    y2 = _tc_stage2(agg1, d0, d1, b1.reshape(1, D), W2)
    agg2 = _sc_scatter(y2, srcflat, dst3, zrows)
    return _tc_stage3(agg2, d0, d1, b2.reshape(1, D))
